# Initial kernel scaffold; baseline (speedup 1.0000x reference)
import jax
import jax.numpy as jnp
from jax.experimental import pallas as pl


def kernel(boxes, box_scores):
    raise NotImplementedError("write your pallas kernel here")


# speedup vs baseline: 7.5481x; 7.5481x over previous
import jax
import jax.numpy as jnp
from jax.experimental import pallas as pl

_N = 20000
_C = 20
_M = 20
_CONF_T = 0.5
_IOU_T = 0.5
_NEG = -1e30


def _nms_dense_kernel(scores_ref, boxes_ref, out_b_ref, out_s_ref):
    y1 = boxes_ref[0:1, :]
    x1 = boxes_ref[1:2, :]
    y2 = boxes_ref[2:3, :]
    x2 = boxes_ref[3:4, :]
    a2 = jnp.maximum(y2 - y1, 0.0) * jnp.maximum(x2 - x1, 0.0)
    lane = jax.lax.broadcasted_iota(jnp.int32, (_C, _N), 1)

    s0 = scores_ref[...]
    s0 = jnp.where(s0 >= _CONF_T, s0, _NEG)

    def step(i, s):
        m = jnp.max(s, axis=1, keepdims=True)
        idx = jnp.min(jnp.where(s == m, lane, _N), axis=1, keepdims=True)
        onehot = lane == idx

        def pick(coord):
            return jnp.sum(jnp.where(onehot, coord, 0.0), axis=1, keepdims=True)

        sy1 = pick(y1)
        sx1 = pick(x1)
        sy2 = pick(y2)
        sx2 = pick(x2)
        sa = jnp.maximum(sy2 - sy1, 0.0) * jnp.maximum(sx2 - sx1, 0.0)
        keep = m > (_NEG * 0.5)

        yy1 = jnp.maximum(sy1, y1)
        xx1 = jnp.maximum(sx1, x1)
        yy2 = jnp.minimum(sy2, y2)
        xx2 = jnp.minimum(sx2, x2)
        inter = jnp.maximum(yy2 - yy1, 0.0) * jnp.maximum(xx2 - xx1, 0.0)
        union = jnp.maximum(sa + a2 - inter, 1e-9)
        iou = inter / union
        suppress = jnp.logical_and(iou > _IOU_T, keep)
        s = jnp.where(jnp.logical_or(suppress, onehot), _NEG, s)

        kf = keep.astype(jnp.float32)
        out_b_ref[i] = jnp.concatenate([sy1, sx1, sy2, sx2], axis=1) * kf
        out_s_ref[i] = jnp.where(keep, m, 0.0)[:, 0]
        return s

    jax.lax.fori_loop(0, _M, step, s0, unroll=False)


def kernel(boxes, box_scores):
    scores_t = box_scores.T
    boxes_t = boxes.T
    out_b, out_s = pl.pallas_call(
        _nms_dense_kernel,
        out_shape=[
            jax.ShapeDtypeStruct((_M, _C, 4), jnp.float32),
            jax.ShapeDtypeStruct((_M, _C), jnp.float32),
        ],
    )(scores_t, boxes_t)
    box_array = out_b.transpose(1, 0, 2).reshape(-1, 4)
    score_array = out_s.T.reshape(-1)
    class_array = jnp.repeat(jnp.arange(_C, dtype=jnp.int32), _M)
    return box_array, score_array, class_array
